# baseline (speedup 1.0000x reference)
import functools

import jax
import jax.numpy as jnp
from jax import lax
from jax.experimental import pallas as pl
from jax.experimental.pallas import tpu as pltpu
from jax.experimental.pallas import tpu_sc as plsc

NC = 2
NS = 16
NW = NC * NS
LANES = 16
CHUNK = 128
PAIR = 2 * CHUNK
PREFETCH = 2


def _sc_aggregate(src_idx, dst_idx, x, n_pad, ep_arr, n_chunks):
  d = x.shape[1]
  rows_per_tile = n_pad // NS

  mesh = plsc.VectorSubcoreMesh(core_axis_name="c", subcore_axis_name="s")

  @functools.partial(
      pl.kernel,
      out_type=[
          jax.ShapeDtypeStruct((NC * n_pad, d), jnp.float32),
          jax.ShapeDtypeStruct((NW, n_pad), jnp.float32),
      ],
      mesh=mesh,
      compiler_params=pltpu.CompilerParams(needs_layout_passes=False),
      scratch_types=[
          pltpu.VMEM((CHUNK,), jnp.int32),
          pltpu.VMEM((CHUNK,), jnp.int32),
          pltpu.VMEM((CHUNK,), jnp.int32),
          pltpu.VMEM((CHUNK,), jnp.int32),
          pltpu.VMEM((CHUNK, d), jnp.float32),
          pltpu.VMEM((CHUNK, d), jnp.float32),
          pltpu.VMEM((n_pad,), jnp.float32),
          pltpu.VMEM_SHARED((n_pad, d), jnp.float32),
          pltpu.SemaphoreType.DMA,
          pltpu.SemaphoreType.DMA,
      ],
  )
  def agg(src_hbm, dst_hbm, x_hbm, summed_out, deg_out,
          idx_s0, idx_s1, idx_d0, idx_d1, rows_0, rows_1, deg_v, accum_sh,
          sem_g0, sem_g1):
    c = lax.axis_index("c")
    s = lax.axis_index("s")
    wid = c * NS + s
    idx_s = [idx_s0, idx_s1]
    idx_d = [idx_d0, idx_d1]
    rows = [rows_0, rows_1]
    sem_g = [sem_g0, sem_g1]

    zero16 = jnp.zeros((LANES,), jnp.float32)
    one16 = jnp.ones((LANES,), jnp.float32)

    def fill_row(i, _):
      def fill_seg(j, _):
        rows_0[i, pl.ds(j * LANES, LANES)] = zero16
        return 0
      lax.fori_loop(0, d // LANES, fill_seg, 0)
      return 0
    lax.fori_loop(0, CHUNK, fill_row, 0)

    def clear_deg(i, _):
      deg_v[pl.ds(i * LANES, LANES)] = zero16
      return 0
    lax.fori_loop(0, n_pad // LANES, clear_deg, 0)

    row0 = s * rows_per_tile
    def clear_blk(i, _):
      pltpu.sync_copy(rows_0, accum_sh.at[pl.ds(row0 + i * CHUNK, CHUNK)])
      return 0
    lax.fori_loop(0, rows_per_tile // CHUNK, clear_blk, 0)

    plsc.subcore_barrier()

    base = wid * ep_arr

    for p in (0, 1):
      pltpu.sync_copy(src_hbm.at[pl.ds(base + p * CHUNK, CHUNK)], idx_s[p])
      pltpu.sync_copy(dst_hbm.at[pl.ds(base + p * CHUNK, CHUNK)], idx_d[p])
      pltpu.async_copy(x_hbm.at[idx_s[p]], rows[p], sem_g[p])

    def pair_body(j2, _):
      for p in (0, 1):
        pltpu.make_async_copy(x_hbm.at[idx_s[p]], rows[p], sem_g[p]).wait()
        pltpu.sync_copy(rows[p], accum_sh.at[idx_d[p]], add=True)
        def vec_body(v, _):
          iv = idx_d[p][pl.ds(v * LANES, LANES)]
          plsc.addupdate_scatter(deg_v, [iv], one16)
          return 0
        lax.fori_loop(0, CHUNK // LANES, vec_body, 0)
        off = base + (j2 * 2 + p + 2) * CHUNK
        pltpu.sync_copy(src_hbm.at[pl.ds(off, CHUNK)], idx_s[p])
        pltpu.sync_copy(dst_hbm.at[pl.ds(off, CHUNK)], idx_d[p])
        pltpu.async_copy(x_hbm.at[idx_s[p]], rows[p], sem_g[p])
      return 0

    lax.fori_loop(0, n_chunks // 2, pair_body, 0)

    for p in (0, 1):
      pltpu.make_async_copy(x_hbm.at[idx_s[p]], rows[p], sem_g[p]).wait()

    plsc.subcore_barrier()

    out_row0 = c * n_pad + row0
    pltpu.sync_copy(accum_sh.at[pl.ds(row0, rows_per_tile)],
                    summed_out.at[pl.ds(out_row0, rows_per_tile)])
    pltpu.sync_copy(deg_v, deg_out.at[wid])

  return agg(src_idx, dst_idx, x)


def _tc_combine(x_pad, summed, degw, w_self, w_neigh, n_pad, blk):
  d = x_pad.shape[1]
  nblk = n_pad // blk

  def body(x_ref, s0_ref, s1_ref, deg_ref, ws_ref, wn_ref, out_ref):
    deg = jnp.sum(deg_ref[...], axis=0)[:, None]
    h = (s0_ref[...] + s1_ref[...]) / jnp.maximum(deg, 1.0)
    out_ref[...] = (
        jnp.dot(x_ref[...], ws_ref[...], preferred_element_type=jnp.float32)
        + jnp.dot(h, wn_ref[...], preferred_element_type=jnp.float32))

  return pl.pallas_call(
      body,
      grid=(nblk,),
      in_specs=[
          pl.BlockSpec((blk, d), lambda i: (i, 0)),
          pl.BlockSpec((blk, d), lambda i: (i, 0)),
          pl.BlockSpec((blk, d), lambda i, nb=nblk: (i + nb, 0)),
          pl.BlockSpec((NW, blk), lambda i: (0, i)),
          pl.BlockSpec((d, d), lambda i: (0, 0)),
          pl.BlockSpec((d, d), lambda i: (0, 0)),
      ],
      out_specs=pl.BlockSpec((blk, d), lambda i: (i, 0)),
      out_shape=jax.ShapeDtypeStruct((n_pad, d), jnp.float32),
  )(x_pad, summed, summed, degw, w_self, w_neigh)


def kernel(x, edge_index, W_self, W_neigh):
  n, d = x.shape
  e = edge_index.shape[1]

  blk = 1024
  n_pad = ((n + blk - 1) // blk) * blk

  ep_raw = e // NW
  ep = ((ep_raw + PAIR - 1) // PAIR) * PAIR
  n_chunks = ep // CHUNK
  ep_arr = ep + PREFETCH * CHUNK
  pad = ep_arr - ep_raw
  src = jnp.pad(edge_index[0].reshape(NW, ep_raw), ((0, 0), (0, pad))).reshape(-1)
  dst = jnp.pad(edge_index[1].reshape(NW, ep_raw), ((0, 0), (0, pad)),
                constant_values=n).reshape(-1)

  summed, degw = _sc_aggregate(src, dst, x, n_pad, ep_arr, n_chunks)

  x_pad = jnp.pad(x, ((0, n_pad - n), (0, 0)))
  out = _tc_combine(x_pad, summed, degw, W_self, W_neigh, n_pad, blk)
  return out[:n]


# speedup vs baseline: 1.9153x; 1.9153x over previous
import functools

import jax
import jax.numpy as jnp
from jax import lax
from jax.experimental import pallas as pl
from jax.experimental.pallas import tpu as pltpu
from jax.experimental.pallas import tpu_sc as plsc

NC = 2
NS = 16
NW = NC * NS
LANES = 16
CHUNK = 128


def _sc_aggregate(src_idx, dst_idx, x, n_pad, ep):
  d = x.shape[1]
  rows_per_tile = n_pad // NS
  n_chunks = ep // CHUNK

  mesh = plsc.VectorSubcoreMesh(core_axis_name="c", subcore_axis_name="s")

  @functools.partial(
      pl.kernel,
      out_type=[
          jax.ShapeDtypeStruct((NC * n_pad, d), jnp.float32),
          jax.ShapeDtypeStruct((NW, n_pad), jnp.float32),
      ],
      mesh=mesh,
      compiler_params=pltpu.CompilerParams(needs_layout_passes=False),
      scratch_types=[
          pltpu.VMEM((CHUNK,), jnp.int32),
          pltpu.VMEM((CHUNK,), jnp.int32),
          pltpu.VMEM((CHUNK, d), jnp.float32),
          pltpu.VMEM((n_pad,), jnp.float32),
          pltpu.VMEM_SHARED((n_pad, d), jnp.float32),
          pltpu.SemaphoreType.DMA,
      ],
  )
  def agg(src_hbm, dst_hbm, x_hbm, summed_out, deg_out,
          idx_s, idx_d, rows, deg_v, accum_sh, sem):
    c = lax.axis_index("c")
    s = lax.axis_index("s")
    wid = c * NS + s

    zero16 = jnp.zeros((LANES,), jnp.float32)
    one16 = jnp.ones((LANES,), jnp.float32)

    def fill_row(i, _):
      def fill_seg(j, _):
        rows[i, pl.ds(j * LANES, LANES)] = zero16
        return 0
      lax.fori_loop(0, d // LANES, fill_seg, 0)
      return 0
    lax.fori_loop(0, CHUNK, fill_row, 0)

    def clear_deg(i, _):
      deg_v[pl.ds(i * LANES, LANES)] = zero16
      return 0
    lax.fori_loop(0, n_pad // LANES, clear_deg, 0)

    row0 = s * rows_per_tile
    def clear_blk(i, _):
      pltpu.sync_copy(rows, accum_sh.at[pl.ds(row0 + i * CHUNK, CHUNK)])
      return 0
    lax.fori_loop(0, rows_per_tile // CHUNK, clear_blk, 0)

    plsc.subcore_barrier()

    base = wid * ep

    def chunk_body(j, _):
      off = base + j * CHUNK
      pltpu.sync_copy(src_hbm.at[pl.ds(off, CHUNK)], idx_s)
      pltpu.sync_copy(dst_hbm.at[pl.ds(off, CHUNK)], idx_d)
      gather = pltpu.async_copy(x_hbm.at[idx_s], rows, sem)
      def vec_body(v, _):
        iv = idx_d[pl.ds(v * LANES, LANES)]
        plsc.addupdate_scatter(deg_v, [iv], one16)
        return 0
      lax.fori_loop(0, CHUNK // LANES, vec_body, 0)
      gather.wait()
      pltpu.sync_copy(rows, accum_sh.at[idx_d], add=True)
      return 0

    lax.fori_loop(0, n_chunks, chunk_body, 0)

    plsc.subcore_barrier()

    out_row0 = c * n_pad + row0
    pltpu.sync_copy(accum_sh.at[pl.ds(row0, rows_per_tile)],
                    summed_out.at[pl.ds(out_row0, rows_per_tile)])
    pltpu.sync_copy(deg_v, deg_out.at[wid])

  return agg(src_idx, dst_idx, x)


def _tc_self(x_pad, w_self, n_pad, blk):
  d = x_pad.shape[1]

  def body(x_ref, ws_ref, out_ref):
    out_ref[...] = jnp.dot(x_ref[...], ws_ref[...],
                           preferred_element_type=jnp.float32)

  return pl.pallas_call(
      body,
      grid=(n_pad // blk,),
      in_specs=[
          pl.BlockSpec((blk, d), lambda i: (i, 0)),
          pl.BlockSpec((d, d), lambda i: (0, 0)),
      ],
      out_specs=pl.BlockSpec((blk, d), lambda i: (i, 0)),
      out_shape=jax.ShapeDtypeStruct((n_pad, d), jnp.float32),
  )(x_pad, w_self)


def _tc_combine(y_self, summed, degw, w_neigh, n_pad, blk):
  d = y_self.shape[1]
  nblk = n_pad // blk

  def body(y_ref, s0_ref, s1_ref, deg_ref, wn_ref, out_ref):
    deg = jnp.sum(deg_ref[...], axis=0)[:, None]
    h = (s0_ref[...] + s1_ref[...]) / jnp.maximum(deg, 1.0)
    out_ref[...] = y_ref[...] + jnp.dot(h, wn_ref[...],
                                        preferred_element_type=jnp.float32)

  return pl.pallas_call(
      body,
      grid=(nblk,),
      in_specs=[
          pl.BlockSpec((blk, d), lambda i: (i, 0)),
          pl.BlockSpec((blk, d), lambda i: (i, 0)),
          pl.BlockSpec((blk, d), lambda i, nb=nblk: (i + nb, 0)),
          pl.BlockSpec((NW, blk), lambda i: (0, i)),
          pl.BlockSpec((d, d), lambda i: (0, 0)),
      ],
      out_specs=pl.BlockSpec((blk, d), lambda i: (i, 0)),
      out_shape=jax.ShapeDtypeStruct((n_pad, d), jnp.float32),
  )(y_self, summed, summed, degw, w_neigh)


def kernel(x, edge_index, W_self, W_neigh):
  n, d = x.shape
  e = edge_index.shape[1]

  blk = 1024
  n_pad = ((n + blk - 1) // blk) * blk

  ep_raw = e // NW
  ep = ((ep_raw + CHUNK - 1) // CHUNK) * CHUNK
  pad = ep - ep_raw
  src = jnp.pad(edge_index[0].reshape(NW, ep_raw), ((0, 0), (0, pad))).reshape(-1)
  dst = jnp.pad(edge_index[1].reshape(NW, ep_raw), ((0, 0), (0, pad)),
                constant_values=n).reshape(-1)

  x_pad = jnp.pad(x, ((0, n_pad - n), (0, 0)))
  y_self = _tc_self(x_pad, W_self, n_pad, blk)

  summed, degw = _sc_aggregate(src, dst, x, n_pad, ep)

  out = _tc_combine(y_self, summed, degw, W_neigh, n_pad, blk)
  return out[:n]
